# baseline (speedup 1.0000x reference)
import functools

import jax
import jax.numpy as jnp
from jax import lax
from jax.experimental import pallas as pl
from jax.experimental.pallas import tpu as pltpu
from jax.experimental.pallas import tpu_sc as plsc

K_ITERS = 10
ALPHA = 0.1

NC = 2
NS = 16
L = 16
CB = 128
NB = 4


def _elu(v):
    return jnp.where(v > 0.0, v, jnp.exp(jnp.minimum(v, 0.0)) - 1.0)


def _rsqrt3(q):
    ti = lax.bitcast_convert_type(q, jnp.int32)
    ti = jnp.int32(0x5F3759DF) - lax.shift_right_logical(ti, 1)
    r = lax.bitcast_convert_type(ti, jnp.float32)
    r = r * (1.5 - 0.5 * q * r * r)
    r = r * (1.5 - 0.5 * q * r * r)
    r = r * (1.5 - 0.5 * q * r * r)
    return r


def _fc_body(x_ref, w1_ref, b1_ref, w2_ref, o_ref, *, ch):
    h = jnp.dot(x_ref[...], w1_ref[...], preferred_element_type=jnp.float32)
    h = h + b1_ref[...]
    g = jnp.dot(h, w2_ref[...], preferred_element_type=jnp.float32)
    o_ref[0] = g[:, :ch]
    o_ref[1] = g[:, ch:]


def _sc_body(g_ref, src_ref, dst_ref, b2_ref, y_ref,
             agg, src_loc, dst_loc, gbuf, rowbuf, nsqe, h0n, b2_loc,
             sg0, sg1, sg2, sg3, ss0, ss1, ss2, ss3,
             *, n, npad, rt, nch, ch):
    c = lax.axis_index("c")
    s = lax.axis_index("s")
    row_lo = s * rt
    base = c * npad
    nh = ch // L
    sgs = [sg0, sg1, sg2, sg3]
    sss = [ss0, ss1, ss2, ss3]

    pltpu.sync_copy(src_ref.at[s], src_loc)
    pltpu.sync_copy(dst_ref.at[s], dst_loc)
    pltpu.sync_copy(b2_ref.at[c], b2_loc)

    def _off_body(i, _):
        for h in range(CB // L):
            sl = pl.ds(h * L, L)
            src_loc[i, sl] = src_loc[i, sl] + base
        return 0
    lax.fori_loop(0, nch, _off_body, 0)

    zv = jnp.zeros((L,), jnp.float32)

    def _fill_gbuf0(v):
        def body(i, _):
            for h in range(nh):
                gbuf[0, i, pl.ds(h * L, L)] = v
            return 0
        lax.fori_loop(0, CB, body, 0)

    def _zero_agg_slice():
        _fill_gbuf0(zv)
        for q in range(rt // CB):
            pltpu.sync_copy(gbuf.at[0], agg.at[pl.ds(row_lo + q * CB, CB)])

    def _edge_pass():
        pltpu.async_copy(y_ref.at[src_loc.at[0]], gbuf.at[0], sg0)
        def body(p, _):
            j0 = 2 * p
            j1 = j0 + 1
            jn = jnp.minimum(j0 + 2, nch - 1)
            pltpu.async_copy(y_ref.at[src_loc.at[j1]], gbuf.at[1], sg1)
            pltpu.make_async_copy(
                y_ref.at[src_loc.at[j0]], gbuf.at[0], sg0).wait()
            pltpu.sync_copy(gbuf.at[0], agg.at[dst_loc.at[j0]], add=True)
            pltpu.async_copy(y_ref.at[src_loc.at[jn]], gbuf.at[0], sg0)
            pltpu.make_async_copy(
                y_ref.at[src_loc.at[j1]], gbuf.at[1], sg1).wait()
            pltpu.sync_copy(gbuf.at[1], agg.at[dst_loc.at[j1]], add=True)
            return 0
        lax.fori_loop(0, nch // 2, body, 0)
        pltpu.make_async_copy(
            y_ref.at[src_loc.at[0]], gbuf.at[0], sg0).wait()
        plsc.subcore_barrier()

    _zero_agg_slice()
    plsc.subcore_barrier()
    _fill_gbuf0(jnp.full((L,), 1.0, jnp.float32))
    def _deg_body(q, _):
        cps = [pltpu.async_copy(
                   gbuf.at[0], agg.at[dst_loc.at[8 * q + l]], ss0, add=True)
               for l in range(8)]
        for cp in cps:
            cp.wait()
        return 0
    lax.fori_loop(0, nch // 8, _deg_body, 0)
    plsc.subcore_barrier()

    pltpu.sync_copy(agg.at[pl.ds(row_lo, rt)], rowbuf)
    _zero_agg_slice()
    def _nsq_body(i, _):
        d = jnp.maximum(rowbuf[i, pl.ds(0, L)], 1.0)
        nsqe[i, pl.ds(0, L)] = (1.0 - ALPHA) / d
        return 0
    lax.fori_loop(0, rt, _nsq_body, 0)

    pltpu.sync_copy(g_ref.at[pl.ds(base + row_lo, rt)], rowbuf)
    def _init_body(i, _):
        q = nsqe[i, pl.ds(0, L)] * (1.0 / (1.0 - ALPHA))
        nbr = q * _rsqrt3(q)
        for h in range(nh):
            sl = pl.ds(h * L, L)
            g = rowbuf[i, sl]
            h0n[i, sl] = ALPHA * g * nbr
            rowbuf[i, sl] = g * nbr
        return 0
    lax.fori_loop(0, rt, _init_body, 0)
    pltpu.sync_copy(rowbuf, y_ref.at[pl.ds(base + row_lo, rt)])
    plsc.subcore_barrier()

    def _iteration(tail):
        _edge_pass()
        pltpu.sync_copy(agg.at[pl.ds(row_lo, rt)], rowbuf)
        _zero_agg_slice()
        def ew(i, _):
            nsqr = nsqe[i, pl.ds(0, L)]
            if tail != 0:
                q = nsqr * (1.0 / (1.0 - ALPHA))
                invr = _rsqrt3(q)
                nbr = q * invr
            for h in range(nh):
                sl = pl.ds(h * L, L)
                y = nsqr * rowbuf[i, sl] + h0n[i, sl]
                if tail == 0:
                    rowbuf[i, sl] = y
                elif tail == 1:
                    u = _elu(y * invr + b2_loc[pl.ds(h * L, L)])
                    h0n[i, sl] = ALPHA * u * nbr
                    rowbuf[i, sl] = u * nbr
                else:
                    rowbuf[i, sl] = _elu(y * invr)
            return 0
        lax.fori_loop(0, rt, ew, 0)
        pltpu.sync_copy(rowbuf, y_ref.at[pl.ds(base + row_lo, rt)])
        plsc.subcore_barrier()

    lax.fori_loop(0, K_ITERS - 1, lambda k, _: (_iteration(0), 0)[1], 0)
    _iteration(1)
    lax.fori_loop(0, K_ITERS - 1, lambda k, _: (_iteration(0), 0)[1], 0)
    _iteration(2)


def kernel(features, edge_index, W1, b1, W2, b2):
    N, D = features.shape
    H = W1.shape[1]
    C = W2.shape[1]
    CH = C // NC
    RT = 640
    NPAD = NS * RT
    NCH = -(-edge_index.shape[1] // (NS * CB))
    NCH = -(-NCH // 8) * 8
    EP = NS * NCH * CB
    E = edge_index.shape[1]

    BM = 1280
    xp = jnp.pad(features, ((0, NPAD - N), (0, 0)))
    g2 = pl.pallas_call(
        functools.partial(_fc_body, ch=CH),
        grid=(NPAD // BM,),
        in_specs=[
            pl.BlockSpec((BM, D), lambda i: (i, 0)),
            pl.BlockSpec((D, H), lambda i: (0, 0)),
            pl.BlockSpec((1, H), lambda i: (0, 0)),
            pl.BlockSpec((H, C), lambda i: (0, 0)),
        ],
        out_specs=pl.BlockSpec((NC, BM, CH), lambda i: (0, i, 0)),
        out_shape=jax.ShapeDtypeStruct((NC, NPAD, CH), jnp.float32),
    )(xp, W1, b1.reshape(1, H), W2)
    g_flat = g2.reshape(NC * NPAD, CH)

    src = edge_index[0]
    dst = edge_index[1]
    src_t = jnp.concatenate(
        [src, jnp.zeros((EP - E,), jnp.int32)]).reshape(NS, NCH, CB)
    dst_t = jnp.concatenate(
        [dst, jnp.full((EP - E,), N, jnp.int32)]).reshape(NS, NCH, CB)
    b2s = b2.reshape(NC, CH)

    mesh = plsc.VectorSubcoreMesh(
        core_axis_name="c", subcore_axis_name="s",
        num_cores=NC, num_subcores=NS)
    sc_fn = pl.kernel(
        functools.partial(_sc_body, n=N, npad=NPAD, rt=RT, nch=NCH, ch=CH),
        out_type=jax.ShapeDtypeStruct((NC * NPAD, CH), jnp.float32),
        mesh=mesh,
        scratch_types=[
            pltpu.VMEM_SHARED((NPAD, CH), jnp.float32),
            pltpu.VMEM((NCH, CB), jnp.int32),
            pltpu.VMEM((NCH, CB), jnp.int32),
            pltpu.VMEM((NB, CB, CH), jnp.float32),
            pltpu.VMEM((RT, CH), jnp.float32),
            pltpu.VMEM((RT, L), jnp.float32),
            pltpu.VMEM((RT, CH), jnp.float32),
            pltpu.VMEM((CH,), jnp.float32),
            pltpu.SemaphoreType.DMA,
            pltpu.SemaphoreType.DMA,
            pltpu.SemaphoreType.DMA,
            pltpu.SemaphoreType.DMA,
            pltpu.SemaphoreType.DMA,
            pltpu.SemaphoreType.DMA,
            pltpu.SemaphoreType.DMA,
            pltpu.SemaphoreType.DMA,
        ],
        compiler_params=pltpu.CompilerParams(use_tc_tiling_on_sc=False),
    )
    yout = sc_fn(g_flat, src_t, dst_t, b2s)
    r = yout.reshape(NC, NPAD, CH)
    return jnp.concatenate([r[0, :N], r[1, :N]], axis=1)


# speedup vs baseline: 1.0003x; 1.0003x over previous
import functools

import jax
import jax.numpy as jnp
from jax import lax
from jax.experimental import pallas as pl
from jax.experimental.pallas import tpu as pltpu
from jax.experimental.pallas import tpu_sc as plsc

K_ITERS = 10
ALPHA = 0.1

NC = 2
NS = 16
L = 16
CB = 128
NB = 4


def _elu(v):
    return jnp.where(v > 0.0, v, jnp.exp(jnp.minimum(v, 0.0)) - 1.0)


def _rsqrt3(q):
    ti = lax.bitcast_convert_type(q, jnp.int32)
    ti = jnp.int32(0x5F3759DF) - lax.shift_right_logical(ti, 1)
    r = lax.bitcast_convert_type(ti, jnp.float32)
    r = r * (1.5 - 0.5 * q * r * r)
    r = r * (1.5 - 0.5 * q * r * r)
    r = r * (1.5 - 0.5 * q * r * r)
    return r


def _fc_body(x_ref, w1_ref, b1_ref, w2_ref, o_ref, *, ch):
    h = jnp.dot(x_ref[...], w1_ref[...], preferred_element_type=jnp.float32)
    h = h + b1_ref[...]
    g = jnp.dot(h, w2_ref[...], preferred_element_type=jnp.float32)
    o_ref[0] = g[:, :ch]
    o_ref[1] = g[:, ch:]


def _sc_body(g_ref, src_ref, dst_ref, b2_ref, y_ref,
             agg, src_loc, dst_loc, gbuf, rowbuf, nsqe, h0n, b2_loc,
             sg0, sg1, sg2, sg3, ss0, ss1, ss2, ss3,
             *, n, npad, rt, nch, ch):
    c = lax.axis_index("c")
    s = lax.axis_index("s")
    row_lo = s * rt
    base = c * npad
    nh = ch // L
    sgs = [sg0, sg1, sg2, sg3]
    sss = [ss0, ss1, ss2, ss3]

    pltpu.sync_copy(src_ref.at[s], src_loc)
    pltpu.sync_copy(dst_ref.at[s], dst_loc)
    pltpu.sync_copy(b2_ref.at[c], b2_loc)

    def _off_body(i, _):
        for h in range(CB // L):
            sl = pl.ds(h * L, L)
            src_loc[i, sl] = src_loc[i, sl] + base
        return 0
    lax.fori_loop(0, nch, _off_body, 0)

    zv = jnp.zeros((L,), jnp.float32)

    def _fill_gbuf0(v):
        def body(i, _):
            for h in range(nh):
                gbuf[0, i, pl.ds(h * L, L)] = v
            return 0
        lax.fori_loop(0, CB, body, 0)

    def _zero_agg_slice():
        _fill_gbuf0(zv)
        for q in range(rt // CB):
            pltpu.sync_copy(gbuf.at[0], agg.at[pl.ds(row_lo + q * CB, CB)])

    def _edge_pass():
        pltpu.async_copy(y_ref.at[src_loc.at[0]], gbuf.at[0], sg0)
        def body(p, _):
            j0 = 2 * p
            j1 = j0 + 1
            jn = jnp.minimum(j0 + 2, nch - 1)
            pltpu.async_copy(y_ref.at[src_loc.at[j1]], gbuf.at[1], sg1)
            pltpu.make_async_copy(
                y_ref.at[src_loc.at[j0]], gbuf.at[0], sg0).wait()
            pltpu.sync_copy(gbuf.at[0], agg.at[dst_loc.at[j0]], add=True)
            pltpu.async_copy(y_ref.at[src_loc.at[jn]], gbuf.at[0], sg0)
            pltpu.make_async_copy(
                y_ref.at[src_loc.at[j1]], gbuf.at[1], sg1).wait()
            pltpu.sync_copy(gbuf.at[1], agg.at[dst_loc.at[j1]], add=True)
            return 0
        lax.fori_loop(0, nch // 2, body, 0)
        pltpu.make_async_copy(
            y_ref.at[src_loc.at[0]], gbuf.at[0], sg0).wait()
        plsc.subcore_barrier()

    with jax.named_scope("deg"):
        _zero_agg_slice()
        plsc.subcore_barrier()
        _fill_gbuf0(jnp.full((L,), 1.0, jnp.float32))
        def _deg_body(q, _):
            cps = [pltpu.async_copy(
                       gbuf.at[0], agg.at[dst_loc.at[8 * q + l]], ss0,
                       add=True)
                   for l in range(8)]
            for cp in cps:
                cp.wait()
            return 0
        lax.fori_loop(0, nch // 8, _deg_body, 0)
        plsc.subcore_barrier()

    pltpu.sync_copy(agg.at[pl.ds(row_lo, rt)], rowbuf)
    _zero_agg_slice()
    def _nsq_body(i, _):
        d = jnp.maximum(rowbuf[i, pl.ds(0, L)], 1.0)
        nsqe[i, pl.ds(0, L)] = (1.0 - ALPHA) / d
        return 0
    lax.fori_loop(0, rt, _nsq_body, 0)

    pltpu.sync_copy(g_ref.at[pl.ds(base + row_lo, rt)], rowbuf)
    def _init_body(i, _):
        q = nsqe[i, pl.ds(0, L)] * (1.0 / (1.0 - ALPHA))
        nbr = q * _rsqrt3(q)
        for h in range(nh):
            sl = pl.ds(h * L, L)
            g = rowbuf[i, sl]
            h0n[i, sl] = ALPHA * g * nbr
            rowbuf[i, sl] = g * nbr
        return 0
    lax.fori_loop(0, rt, _init_body, 0)
    pltpu.sync_copy(rowbuf, y_ref.at[pl.ds(base + row_lo, rt)])
    plsc.subcore_barrier()

    def _iteration(tail):
        with jax.named_scope("edges"):
            _edge_pass()
        with jax.named_scope("stage"):
            pltpu.sync_copy(agg.at[pl.ds(row_lo, rt)], rowbuf)
            _zero_agg_slice()
        def ew(i, _):
            nsqr = nsqe[i, pl.ds(0, L)]
            if tail != 0:
                q = nsqr * (1.0 / (1.0 - ALPHA))
                invr = _rsqrt3(q)
                nbr = q * invr
            for h in range(nh):
                sl = pl.ds(h * L, L)
                y = nsqr * rowbuf[i, sl] + h0n[i, sl]
                if tail == 0:
                    rowbuf[i, sl] = y
                elif tail == 1:
                    u = _elu(y * invr + b2_loc[pl.ds(h * L, L)])
                    h0n[i, sl] = ALPHA * u * nbr
                    rowbuf[i, sl] = u * nbr
                else:
                    rowbuf[i, sl] = _elu(y * invr)
            return 0
        with jax.named_scope("ew"):
            lax.fori_loop(0, rt, ew, 0)
        with jax.named_scope("ywrite"):
            pltpu.sync_copy(rowbuf, y_ref.at[pl.ds(base + row_lo, rt)])
            plsc.subcore_barrier()

    lax.fori_loop(0, K_ITERS - 1, lambda k, _: (_iteration(0), 0)[1], 0)
    _iteration(1)
    lax.fori_loop(0, K_ITERS - 1, lambda k, _: (_iteration(0), 0)[1], 0)
    _iteration(2)


def kernel(features, edge_index, W1, b1, W2, b2):
    N, D = features.shape
    H = W1.shape[1]
    C = W2.shape[1]
    CH = C // NC
    RT = 640
    NPAD = NS * RT
    NCH = -(-edge_index.shape[1] // (NS * CB))
    NCH = -(-NCH // 8) * 8
    EP = NS * NCH * CB
    E = edge_index.shape[1]

    BM = 1280
    xp = jnp.pad(features, ((0, NPAD - N), (0, 0)))
    g2 = pl.pallas_call(
        functools.partial(_fc_body, ch=CH),
        grid=(NPAD // BM,),
        in_specs=[
            pl.BlockSpec((BM, D), lambda i: (i, 0)),
            pl.BlockSpec((D, H), lambda i: (0, 0)),
            pl.BlockSpec((1, H), lambda i: (0, 0)),
            pl.BlockSpec((H, C), lambda i: (0, 0)),
        ],
        out_specs=pl.BlockSpec((NC, BM, CH), lambda i: (0, i, 0)),
        out_shape=jax.ShapeDtypeStruct((NC, NPAD, CH), jnp.float32),
    )(xp, W1, b1.reshape(1, H), W2)
    g_flat = g2.reshape(NC * NPAD, CH)

    src = edge_index[0]
    dst = edge_index[1]
    src_t = jnp.concatenate(
        [src, jnp.zeros((EP - E,), jnp.int32)]).reshape(NS, NCH, CB)
    dst_t = jnp.concatenate(
        [dst, jnp.full((EP - E,), N, jnp.int32)]).reshape(NS, NCH, CB)
    b2s = b2.reshape(NC, CH)

    mesh = plsc.VectorSubcoreMesh(
        core_axis_name="c", subcore_axis_name="s",
        num_cores=NC, num_subcores=NS)
    sc_fn = pl.kernel(
        functools.partial(_sc_body, n=N, npad=NPAD, rt=RT, nch=NCH, ch=CH),
        out_type=jax.ShapeDtypeStruct((NC * NPAD, CH), jnp.float32),
        mesh=mesh,
        scratch_types=[
            pltpu.VMEM_SHARED((NPAD, CH), jnp.float32),
            pltpu.VMEM((NCH, CB), jnp.int32),
            pltpu.VMEM((NCH, CB), jnp.int32),
            pltpu.VMEM((NB, CB, CH), jnp.float32),
            pltpu.VMEM((RT, CH), jnp.float32),
            pltpu.VMEM((RT, L), jnp.float32),
            pltpu.VMEM((RT, CH), jnp.float32),
            pltpu.VMEM((CH,), jnp.float32),
            pltpu.SemaphoreType.DMA,
            pltpu.SemaphoreType.DMA,
            pltpu.SemaphoreType.DMA,
            pltpu.SemaphoreType.DMA,
            pltpu.SemaphoreType.DMA,
            pltpu.SemaphoreType.DMA,
            pltpu.SemaphoreType.DMA,
            pltpu.SemaphoreType.DMA,
        ],
        compiler_params=pltpu.CompilerParams(use_tc_tiling_on_sc=False),
    )
    yout = sc_fn(g_flat, src_t, dst_t, b2s)
    r = yout.reshape(NC, NPAD, CH)
    return jnp.concatenate([r[0, :N], r[1, :N]], axis=1)


# speedup vs baseline: 1.5082x; 1.5078x over previous
import functools

import jax
import jax.numpy as jnp
from jax import lax
from jax.experimental import pallas as pl
from jax.experimental.pallas import tpu as pltpu
from jax.experimental.pallas import tpu_sc as plsc

K_ITERS = 10
ALPHA = 0.1

NC = 2
NS = 16
L = 16
CB = 128


def _elu(v):
    return jnp.where(v > 0.0, v, jnp.exp(jnp.minimum(v, 0.0)) - 1.0)


def _fc_body(x_ref, w1_ref, b1_ref, w2_ref, o_ref, *, ch):
    h = jnp.dot(x_ref[...], w1_ref[...], preferred_element_type=jnp.float32)
    h = h + b1_ref[...]
    g = jnp.dot(h, w2_ref[...], preferred_element_type=jnp.float32)
    o_ref[0] = g[:, :ch]
    o_ref[1] = g[:, ch:]


def _sc_body(g_ref, src_ref, dst_ref, b2_ref, y_ref,
             agg, src_loc, dst_loc, gbuf, rowbuf, normb, h0n, b2_loc,
             sem0, sem1,
             *, n, npad, rt, nch, ch):
    c = lax.axis_index("c")
    s = lax.axis_index("s")
    row_lo = s * rt
    base = c * npad
    nh = ch // L

    pltpu.sync_copy(src_ref.at[s], src_loc)
    pltpu.sync_copy(dst_ref.at[s], dst_loc)
    pltpu.sync_copy(b2_ref.at[c], b2_loc)

    def _off_body(i, _):
        for h in range(CB // L):
            sl = pl.ds(h * L, L)
            src_loc[i, sl] = src_loc[i, sl] + base
        return 0
    lax.fori_loop(0, nch, _off_body, 0)

    zv = jnp.zeros((L,), jnp.float32)

    def _fill_gbuf0(v):
        def body(i, _):
            for h in range(nh):
                gbuf[0, i, pl.ds(h * L, L)] = v
            return 0
        lax.fori_loop(0, CB, body, 0)

    def _zero_agg_slice():
        _fill_gbuf0(zv)
        for q in range(rt // CB):
            pltpu.sync_copy(gbuf.at[0], agg.at[pl.ds(row_lo + q * CB, CB)])

    def _edge_pass():
        pltpu.async_copy(y_ref.at[src_loc.at[0]], gbuf.at[0], sem0)
        def body(p, _):
            j0 = 2 * p
            j1 = j0 + 1
            jn = jnp.minimum(j0 + 2, nch - 1)
            pltpu.async_copy(y_ref.at[src_loc.at[j1]], gbuf.at[1], sem1)
            pltpu.make_async_copy(
                y_ref.at[src_loc.at[j0]], gbuf.at[0], sem0).wait()
            pltpu.sync_copy(gbuf.at[0], agg.at[dst_loc.at[j0]], add=True)
            pltpu.async_copy(y_ref.at[src_loc.at[jn]], gbuf.at[0], sem0)
            pltpu.make_async_copy(
                y_ref.at[src_loc.at[j1]], gbuf.at[1], sem1).wait()
            pltpu.sync_copy(gbuf.at[1], agg.at[dst_loc.at[j1]], add=True)
            return 0
        lax.fori_loop(0, nch // 2, body, 0)
        pltpu.make_async_copy(
            y_ref.at[src_loc.at[0]], gbuf.at[0], sem0).wait()
        plsc.subcore_barrier()

    _zero_agg_slice()
    plsc.subcore_barrier()
    _fill_gbuf0(jnp.full((L,), 1.0, jnp.float32))
    def _deg_body(j, _):
        pltpu.sync_copy(gbuf.at[0], agg.at[dst_loc.at[j]], add=True)
        return 0
    lax.fori_loop(0, nch, _deg_body, 0)
    plsc.subcore_barrier()

    pltpu.sync_copy(agg.at[pl.ds(row_lo, rt)], rowbuf)
    _zero_agg_slice()
    def _norm_body(i, _):
        for h in range(nh):
            sl = pl.ds(h * L, L)
            d = jnp.maximum(rowbuf[i, sl], 1.0)
            ti = lax.bitcast_convert_type(d, jnp.int32)
            ti = jnp.int32(0x5F3759DF) - lax.shift_right_logical(ti, 1)
            nb = lax.bitcast_convert_type(ti, jnp.float32)
            nb = nb * (1.5 - 0.5 * d * nb * nb)
            nb = nb * (1.5 - 0.5 * d * nb * nb)
            nb = nb * (1.5 - 0.5 * d * nb * nb)
            normb[i, sl] = nb
        return 0
    lax.fori_loop(0, rt, _norm_body, 0)

    pltpu.sync_copy(g_ref.at[pl.ds(base + row_lo, rt)], rowbuf)
    def _init_body(i, _):
        for h in range(nh):
            sl = pl.ds(h * L, L)
            g = rowbuf[i, sl]
            nb = normb[i, sl]
            h0n[i, sl] = ALPHA * g * nb
            rowbuf[i, sl] = g * nb
        return 0
    lax.fori_loop(0, rt, _init_body, 0)
    pltpu.sync_copy(rowbuf, y_ref.at[pl.ds(base + row_lo, rt)])
    plsc.subcore_barrier()

    def _iteration(tail):
        _edge_pass()
        pltpu.sync_copy(agg.at[pl.ds(row_lo, rt)], rowbuf)
        _zero_agg_slice()
        def ew(i, _):
            for h in range(nh):
                sl = pl.ds(h * L, L)
                a = rowbuf[i, sl]
                nb = normb[i, sl]
                y = (1.0 - ALPHA) * nb * nb * a + h0n[i, sl]
                if tail == 0:
                    rowbuf[i, sl] = y
                elif tail == 1:
                    u = _elu(y / nb + b2_loc[pl.ds(h * L, L)])
                    h0n[i, sl] = ALPHA * u * nb
                    rowbuf[i, sl] = u * nb
                else:
                    rowbuf[i, sl] = _elu(y / nb)
            return 0
        lax.fori_loop(0, rt, ew, 0)
        pltpu.sync_copy(rowbuf, y_ref.at[pl.ds(base + row_lo, rt)])
        plsc.subcore_barrier()

    lax.fori_loop(0, K_ITERS - 1, lambda k, _: (_iteration(0), 0)[1], 0)
    _iteration(1)
    lax.fori_loop(0, K_ITERS - 1, lambda k, _: (_iteration(0), 0)[1], 0)
    _iteration(2)


def kernel(features, edge_index, W1, b1, W2, b2):
    N, D = features.shape
    H = W1.shape[1]
    C = W2.shape[1]
    CH = C // NC
    RT = 640
    NPAD = NS * RT
    NCH = -(-edge_index.shape[1] // (NS * CB))
    NCH += NCH % 2
    EP = NS * NCH * CB
    E = edge_index.shape[1]

    BM = 1280
    xp = jnp.pad(features, ((0, NPAD - N), (0, 0)))
    g2 = pl.pallas_call(
        functools.partial(_fc_body, ch=CH),
        grid=(NPAD // BM,),
        in_specs=[
            pl.BlockSpec((BM, D), lambda i: (i, 0)),
            pl.BlockSpec((D, H), lambda i: (0, 0)),
            pl.BlockSpec((1, H), lambda i: (0, 0)),
            pl.BlockSpec((H, C), lambda i: (0, 0)),
        ],
        out_specs=pl.BlockSpec((NC, BM, CH), lambda i: (0, i, 0)),
        out_shape=jax.ShapeDtypeStruct((NC, NPAD, CH), jnp.float32),
    )(xp, W1, b1.reshape(1, H), W2)
    g_flat = g2.reshape(NC * NPAD, CH)

    src = edge_index[0]
    dst = edge_index[1]
    pad_dst = N + jnp.arange(EP - E, dtype=jnp.int32) % (NPAD - N)
    src_t = jnp.concatenate(
        [src, jnp.zeros((EP - E,), jnp.int32)]).reshape(NS, NCH, CB)
    dst_t = jnp.concatenate([dst, pad_dst]).reshape(NS, NCH, CB)
    b2s = b2.reshape(NC, CH)

    mesh = plsc.VectorSubcoreMesh(
        core_axis_name="c", subcore_axis_name="s",
        num_cores=NC, num_subcores=NS)
    sc_fn = pl.kernel(
        functools.partial(_sc_body, n=N, npad=NPAD, rt=RT, nch=NCH, ch=CH),
        out_type=jax.ShapeDtypeStruct((NC * NPAD, CH), jnp.float32),
        mesh=mesh,
        scratch_types=[
            pltpu.VMEM_SHARED((NPAD, CH), jnp.float32),
            pltpu.VMEM((NCH, CB), jnp.int32),
            pltpu.VMEM((NCH, CB), jnp.int32),
            pltpu.VMEM((2, CB, CH), jnp.float32),
            pltpu.VMEM((RT, CH), jnp.float32),
            pltpu.VMEM((RT, CH), jnp.float32),
            pltpu.VMEM((RT, CH), jnp.float32),
            pltpu.VMEM((CH,), jnp.float32),
            pltpu.SemaphoreType.DMA,
            pltpu.SemaphoreType.DMA,
        ],
        compiler_params=pltpu.CompilerParams(use_tc_tiling_on_sc=False),
    )
    yout = sc_fn(g_flat, src_t, dst_t, b2s)
    r = yout.reshape(NC, NPAD, CH)
    return jnp.concatenate([r[0, :N], r[1, :N]], axis=1)
